# Initial kernel scaffold; baseline (speedup 1.0000x reference)
import jax
import jax.numpy as jnp
from jax.experimental import pallas as pl


def kernel(hidden_states, q_w, q_b, k_w, k_b, v_w, v_b, o_w, o_b):
    raise NotImplementedError("write your pallas kernel here")


# speedup vs baseline: 1.8528x; 1.8528x over previous
import functools

import jax
import jax.numpy as jnp
from jax import lax
from jax.experimental import pallas as pl
from jax.experimental.pallas import tpu as pltpu

_VMEM_LIMIT = 48 * 1024 * 1024
_LOG2E = 1.4426950408889634


def _qkv_kernel(x_ref, wq_ref, wk_ref, wv_ref, bq_ref, bk_ref, bv_ref,
                q_ref, k_ref, v_ref):
    x = x_ref[...].astype(jnp.bfloat16)
    q = jnp.dot(x, wq_ref[...], preferred_element_type=jnp.float32)
    q_ref[...] = (q + bq_ref[...]).astype(q_ref.dtype)
    k = jnp.dot(x, wk_ref[...], preferred_element_type=jnp.float32)
    k_ref[...] = (k + bk_ref[...]).astype(k_ref.dtype)
    v = jnp.dot(x, wv_ref[...], preferred_element_type=jnp.float32)
    v_ref[...] = (v + bv_ref[...]).astype(v_ref.dtype)


def _qkv_proj(x, wq, bq, wk, bk, wv, bv):
    M, K = x.shape
    N = wq.shape[1]
    TM = 512
    grid = (M // TM,)

    x_spec = pl.BlockSpec((TM, K), lambda i: (i, 0))
    w_spec = pl.BlockSpec((K, N), lambda i: (0, 0))
    b_spec = pl.BlockSpec((1, N), lambda i: (0, 0))
    o_spec = pl.BlockSpec((TM, N), lambda i: (i, 0))

    return pl.pallas_call(
        _qkv_kernel,
        out_shape=(jax.ShapeDtypeStruct((M, N), jnp.bfloat16),) * 3,
        grid=grid,
        in_specs=[x_spec, w_spec, w_spec, w_spec, b_spec, b_spec, b_spec],
        out_specs=[o_spec, o_spec, o_spec],
        compiler_params=pltpu.CompilerParams(
            dimension_semantics=("parallel",),
            vmem_limit_bytes=_VMEM_LIMIT),
    )(x, wq, wk, wv, bq.reshape(1, N), bk.reshape(1, N), bv.reshape(1, N))


def _attn_kernel(q_ref, k_ref, v_ref, o_ref, m_sc, l_sc, acc_sc):
    qi = pl.program_id(1)
    ki = pl.program_id(2)

    @pl.when(ki == 0)
    def _():
        m_sc[...] = jnp.full_like(m_sc, -1e30)
        l_sc[...] = jnp.zeros_like(l_sc)
        acc_sc[...] = jnp.zeros_like(acc_sc)

    def process(masked):
        s = lax.dot_general(q_ref[...], k_ref[...],
                            dimension_numbers=(((1,), (1,)), ((), ())),
                            preferred_element_type=jnp.float32)
        if masked:
            row = lax.broadcasted_iota(jnp.int32, s.shape, 0)
            col = lax.broadcasted_iota(jnp.int32, s.shape, 1)
            s = jnp.where(col <= row, s, jnp.float32(-1e30))
        m_prev = m_sc[...]
        m_new = jnp.maximum(m_prev, jnp.max(s, axis=-1, keepdims=True))
        alpha = jnp.exp2(m_prev - m_new)
        p = jnp.exp2(s - m_new)
        l_sc[...] = alpha * l_sc[...] + jnp.sum(p, axis=-1, keepdims=True)
        acc_sc[...] = alpha * acc_sc[...] + jnp.dot(
            p.astype(jnp.bfloat16), v_ref[...],
            preferred_element_type=jnp.float32)
        m_sc[...] = m_new

    @pl.when(ki < qi)
    def _():
        process(masked=False)

    @pl.when(ki == qi)
    def _():
        process(masked=True)
        o_ref[...] = (acc_sc[...] / l_sc[...]).astype(o_ref.dtype)


def _flash_attention(q, k, v):
    BH, T, Dh = q.shape
    TS = 512 if T % 512 == 0 else 256
    n_blk = T // TS
    grid = (BH, n_blk, n_blk)

    q_spec = pl.BlockSpec((None, TS, Dh), lambda b, qi, ki: (b, qi, 0))
    kv_spec = pl.BlockSpec((None, TS, Dh),
                           lambda b, qi, ki: (b, jnp.minimum(ki, qi), 0))

    return pl.pallas_call(
        _attn_kernel,
        out_shape=jax.ShapeDtypeStruct((BH, T, Dh), jnp.bfloat16),
        grid=grid,
        in_specs=[q_spec, kv_spec, kv_spec],
        out_specs=q_spec,
        scratch_shapes=[
            pltpu.VMEM((TS, 1), jnp.float32),
            pltpu.VMEM((TS, 1), jnp.float32),
            pltpu.VMEM((TS, Dh), jnp.float32),
        ],
        compiler_params=pltpu.CompilerParams(
            dimension_semantics=("parallel", "parallel", "arbitrary"),
            vmem_limit_bytes=_VMEM_LIMIT),
    )(q, k, v)


def _out_kernel(x_ref, w_ref, b_ref, o_ref):
    acc = jnp.dot(x_ref[...], w_ref[...], preferred_element_type=jnp.float32)
    o_ref[...] = (acc + b_ref[...]).astype(o_ref.dtype)


def _out_proj(x, w, b):
    M, K = x.shape
    N = w.shape[1]
    TM = 512
    grid = (M // TM,)

    return pl.pallas_call(
        _out_kernel,
        out_shape=jax.ShapeDtypeStruct((M, N), jnp.float32),
        grid=grid,
        in_specs=[
            pl.BlockSpec((TM, K), lambda i: (i, 0)),
            pl.BlockSpec((K, N), lambda i: (0, 0)),
            pl.BlockSpec((1, N), lambda i: (0, 0)),
        ],
        out_specs=pl.BlockSpec((TM, N), lambda i: (i, 0)),
        compiler_params=pltpu.CompilerParams(
            dimension_semantics=("parallel",),
            vmem_limit_bytes=_VMEM_LIMIT),
    )(x, w, b.reshape(1, N))


def kernel(hidden_states, q_w, q_b, k_w, k_b, v_w, v_b, o_w, o_b):
    B, T, E = hidden_states.shape
    H = 16
    Dh = E // H
    scale = float(Dh) ** -0.5 * _LOG2E

    x2d = hidden_states.reshape(B * T, E)
    wq = (q_w * scale).astype(jnp.bfloat16)
    bq = q_b * scale
    q, k, v = _qkv_proj(x2d, wq, bq,
                        k_w.astype(jnp.bfloat16), k_b,
                        v_w.astype(jnp.bfloat16), v_b)

    def to_heads(t):
        return (t.reshape(B, T, H, Dh).transpose(0, 2, 1, 3)
                 .reshape(B * H, T, Dh))

    o = _flash_attention(to_heads(q), to_heads(k), to_heads(v))
    o2d = (o.reshape(B, H, T, Dh).transpose(0, 2, 1, 3).reshape(B * T, E))
    out = _out_proj(o2d, o_w.astype(jnp.bfloat16), o_b)
    return out.reshape(B, T, E)


# speedup vs baseline: 8.2238x; 4.4386x over previous
import functools

import jax
import jax.numpy as jnp
from jax import lax
from jax.experimental import pallas as pl
from jax.experimental.pallas import tpu as pltpu

_VMEM_LIMIT = 60 * 1024 * 1024
_LOG2E = 1.4426950408889634
_NEG = -1e30


def _fused_kernel(x_ref, wq_ref, wk_ref, wv_ref, wo_ref,
                  bq_ref, bk_ref, bv_ref, bo_ref, out_ref,
                  q_sc, k_sc, v_sc, o_sc, *, nh, dh):
    T, E = x_ref.shape

    x = x_ref[...].astype(jnp.bfloat16)
    q_sc[...] = (jnp.dot(x, wq_ref[...], preferred_element_type=jnp.float32)
                 + bq_ref[...]).astype(jnp.bfloat16)
    k_sc[...] = (jnp.dot(x, wk_ref[...], preferred_element_type=jnp.float32)
                 + bk_ref[...]).astype(jnp.bfloat16)
    v_sc[...] = (jnp.dot(x, wv_ref[...], preferred_element_type=jnp.float32)
                 + bv_ref[...]).astype(jnp.bfloat16)

    TQ = T // 2
    row = lax.broadcasted_iota(jnp.int32, (TQ, TQ), 0)
    col = lax.broadcasted_iota(jnp.int32, (TQ, TQ), 1)
    diag_mask = jnp.where(col <= row, jnp.float32(0), jnp.float32(_NEG))

    nt = (((1,), (1,)), ((), ()))

    for h in range(nh):
        sl = pl.ds(h * dh, dh)
        q0 = q_sc[0:TQ, sl]
        q1 = q_sc[TQ:T, sl]
        k0 = k_sc[0:TQ, sl]
        k1 = k_sc[TQ:T, sl]
        v0 = v_sc[0:TQ, sl]
        v1 = v_sc[TQ:T, sl]

        s00 = lax.dot_general(q0, k0, nt,
                              preferred_element_type=jnp.float32) + diag_mask
        m0 = jnp.max(s00, axis=-1, keepdims=True)
        p00 = jnp.exp2(s00 - m0)
        l0 = jnp.sum(p00, axis=-1, keepdims=True)
        o0 = jnp.dot(p00.astype(jnp.bfloat16), v0,
                     preferred_element_type=jnp.float32) / l0
        o_sc[0:TQ, sl] = o0.astype(jnp.bfloat16)

        s10 = lax.dot_general(q1, k0, nt, preferred_element_type=jnp.float32)
        s11 = lax.dot_general(q1, k1, nt,
                              preferred_element_type=jnp.float32) + diag_mask
        m1 = jnp.maximum(jnp.max(s10, axis=-1, keepdims=True),
                         jnp.max(s11, axis=-1, keepdims=True))
        p10 = jnp.exp2(s10 - m1)
        p11 = jnp.exp2(s11 - m1)
        l1 = (jnp.sum(p10, axis=-1, keepdims=True)
              + jnp.sum(p11, axis=-1, keepdims=True))
        o1 = (jnp.dot(p10.astype(jnp.bfloat16), v0,
                      preferred_element_type=jnp.float32)
              + jnp.dot(p11.astype(jnp.bfloat16), v1,
                        preferred_element_type=jnp.float32)) / l1
        o_sc[TQ:T, sl] = o1.astype(jnp.bfloat16)

    out = jnp.dot(o_sc[...], wo_ref[...], preferred_element_type=jnp.float32)
    out_ref[...] = out + bo_ref[...]


def kernel(hidden_states, q_w, q_b, k_w, k_b, v_w, v_b, o_w, o_b):
    B, T, E = hidden_states.shape
    H = 16
    Dh = E // H
    scale = float(Dh) ** -0.5 * _LOG2E

    wq = (q_w * scale).astype(jnp.bfloat16)
    bq = (q_b * scale).reshape(1, E)

    x_spec = pl.BlockSpec((None, T, E), lambda b: (b, 0, 0))
    w_spec = pl.BlockSpec((E, E), lambda b: (0, 0))
    b_spec = pl.BlockSpec((1, E), lambda b: (0, 0))

    out = pl.pallas_call(
        functools.partial(_fused_kernel, nh=H, dh=Dh),
        out_shape=jax.ShapeDtypeStruct((B, T, E), jnp.float32),
        grid=(B,),
        in_specs=[x_spec, w_spec, w_spec, w_spec, w_spec,
                  b_spec, b_spec, b_spec, b_spec],
        out_specs=x_spec,
        scratch_shapes=[
            pltpu.VMEM((T, E), jnp.bfloat16),
            pltpu.VMEM((T, E), jnp.bfloat16),
            pltpu.VMEM((T, E), jnp.bfloat16),
            pltpu.VMEM((T, E), jnp.bfloat16),
        ],
        compiler_params=pltpu.CompilerParams(
            dimension_semantics=("parallel",),
            vmem_limit_bytes=_VMEM_LIMIT),
    )(hidden_states, wq,
      k_w.astype(jnp.bfloat16), v_w.astype(jnp.bfloat16),
      o_w.astype(jnp.bfloat16),
      bq, k_b.reshape(1, E), v_b.reshape(1, E), o_b.reshape(1, E))
    return out
